# Initial kernel scaffold; baseline (speedup 1.0000x reference)
import jax
import jax.numpy as jnp
from jax.experimental import pallas as pl


def kernel(x, edge_index, W1, b1, W2, b2, Wfc, bfc):
    raise NotImplementedError("write your pallas kernel here")


# speedup vs baseline: 12.3002x; 12.3002x over previous
import functools
import jax
import jax.numpy as jnp
from jax import lax
from jax.experimental import pallas as pl
from jax.experimental.pallas import tpu as pltpu
from jax.experimental.pallas import tpu_sc as plsc

NC = 2
NS = 16
NW = NC * NS
CHUNK = 128

N = 10000
D = 128
NPAD = 10240
ROWS_PER_TILE = NPAD // NS
DEGW = 128


def _mesh():
    return plsc.VectorSubcoreMesh(
        core_axis_name="c", subcore_axis_name="s", num_cores=NC, num_subcores=NS
    )


def _deg_body(nchunks, dst_hbm, out_hbm, idx_d, buf_v, deg_sh):
    c = lax.axis_index("c")
    s = lax.axis_index("s")
    wid = s * NC + c

    def fill(val):
        def body(r, _):
            for k in range(DEGW // 16):
                buf_v[r, pl.ds(k * 16, 16)] = jnp.full((16,), val, jnp.float32)
            return _
        lax.fori_loop(0, CHUNK, body, None)

    fill(0.0)
    for i in range(ROWS_PER_TILE // CHUNK):
        pltpu.sync_copy(buf_v, deg_sh.at[pl.ds(s * ROWS_PER_TILE + i * CHUNK, CHUNK)])
    fill(1.0)
    plsc.subcore_barrier()

    pltpu.sync_copy(dst_hbm.at[wid], idx_d)

    def step(j, _):
        pltpu.sync_copy(buf_v, deg_sh.at[idx_d.at[j]], add=True)
        return _

    lax.fori_loop(0, nchunks, step, None)
    plsc.subcore_barrier()

    for i in range(ROWS_PER_TILE // CHUNK):
        base = s * ROWS_PER_TILE + i * CHUNK
        pltpu.sync_copy(deg_sh.at[pl.ds(base, CHUNK)], buf_v)
        pltpu.sync_copy(buf_v, out_hbm.at[c, pl.ds(base, CHUNK)])


def _sc_deg(dst_p):
    nchunks = dst_p.shape[1]
    body = functools.partial(_deg_body, nchunks)
    return pl.kernel(
        body,
        out_type=jax.ShapeDtypeStruct((NC, NPAD, DEGW), jnp.float32),
        mesh=_mesh(),
        scratch_types=[
            pltpu.VMEM((nchunks, CHUNK), jnp.int32),
            pltpu.VMEM((CHUNK, DEGW), jnp.float32),
            pltpu.VMEM_SHARED((NPAD, DEGW), jnp.float32),
        ],
    )(dst_p)


def _agg_body(nchunks, g_hbm, src_hbm, dst_hbm, out_hbm,
              idx_s, idx_d, rows_v, agg_sh, sem):
    c = lax.axis_index("c")
    s = lax.axis_index("s")
    wid = s * NC + c

    def init(r, _):
        for k in range(D // 16):
            rows_v[r, pl.ds(k * 16, 16)] = jnp.zeros((16,), jnp.float32)
        return _

    lax.fori_loop(0, CHUNK, init, None)
    for i in range(ROWS_PER_TILE // CHUNK):
        pltpu.sync_copy(rows_v, agg_sh.at[pl.ds(s * ROWS_PER_TILE + i * CHUNK, CHUNK)])
    plsc.subcore_barrier()

    pltpu.sync_copy(src_hbm.at[wid], idx_s)
    pltpu.sync_copy(dst_hbm.at[wid], idx_d)

    def step(j, _):
        pltpu.async_copy(g_hbm.at[idx_s.at[j]], rows_v, sem).wait()
        pltpu.sync_copy(rows_v, agg_sh.at[idx_d.at[j]], add=True)
        return _

    lax.fori_loop(0, nchunks, step, None)
    plsc.subcore_barrier()

    for i in range(ROWS_PER_TILE // CHUNK):
        base = s * ROWS_PER_TILE + i * CHUNK
        pltpu.sync_copy(agg_sh.at[pl.ds(base, CHUNK)], rows_v)
        pltpu.sync_copy(rows_v, out_hbm.at[c, pl.ds(base, CHUNK)])


def _sc_agg(g, src_p, dst_p):
    nchunks = src_p.shape[1]
    body = functools.partial(_agg_body, nchunks)
    return pl.kernel(
        body,
        out_type=jax.ShapeDtypeStruct((NC, NPAD, D), jnp.float32),
        mesh=_mesh(),
        scratch_types=[
            pltpu.VMEM((nchunks, CHUNK), jnp.int32),
            pltpu.VMEM((nchunks, CHUNK), jnp.int32),
            pltpu.VMEM((CHUNK, D), jnp.float32),
            pltpu.VMEM_SHARED((NPAD, D), jnp.float32),
            pltpu.SemaphoreType.DMA,
        ],
    )(g, src_p, dst_p)


def _prep_body(deg_ref, x_ref, w1_ref, dinv_ref, g_ref):
    deg = deg_ref[0, 0:N, 0:1] + deg_ref[1, 0:N, 0:1] + 1.0
    dinv = lax.rsqrt(deg)
    dinv_ref[...] = dinv
    h = jnp.dot(x_ref[...], w1_ref[...], preferred_element_type=jnp.float32)
    g_ref[...] = dinv * h


def _tc_prep(degp, x, W1):
    return pl.pallas_call(
        _prep_body,
        out_shape=(
            jax.ShapeDtypeStruct((N, 1), jnp.float32),
            jax.ShapeDtypeStruct((N, D), jnp.float32),
        ),
    )(degp, x, W1)


def _mid_body(aggp_ref, g1_ref, dinv_ref, b1_ref, w2_ref, g2_ref):
    agg = aggp_ref[0, 0:N, :] + aggp_ref[1, 0:N, :] + g1_ref[...]
    h1 = jnp.maximum(dinv_ref[...] * agg + b1_ref[...], 0.0)
    h2 = jnp.dot(h1, w2_ref[...], preferred_element_type=jnp.float32)
    g2_ref[...] = dinv_ref[...] * h2


def _tc_mid(aggp, g1, dinv, b1, W2):
    return pl.pallas_call(
        _mid_body,
        out_shape=jax.ShapeDtypeStruct((N, D), jnp.float32),
    )(aggp, g1, dinv, b1, W2)


def _head_body(aggp_ref, g2_ref, dinv_ref, b2_ref, wfc_ref, bfc_ref, o_ref):
    agg = aggp_ref[0, 0:N, :] + aggp_ref[1, 0:N, :] + g2_ref[...]
    h2 = jnp.maximum(dinv_ref[...] * agg + b2_ref[...], 0.0)
    m = jnp.mean(h2, axis=0, keepdims=True)
    z = jnp.dot(m, wfc_ref[...], preferred_element_type=jnp.float32) + bfc_ref[...]
    o_ref[...] = jax.nn.sigmoid(z)


def _tc_head(aggp, g2, dinv, b2, Wfc, bfc):
    return pl.pallas_call(
        _head_body,
        out_shape=jax.ShapeDtypeStruct((1, 1), jnp.float32),
    )(aggp, g2, dinv, b2, Wfc, bfc)


def kernel(x, edge_index, W1, b1, W2, b2, Wfc, bfc):
    src = edge_index[0].astype(jnp.int32)
    dst = edge_index[1].astype(jnp.int32)
    e = src.shape[0]
    epad = -(-e // (NW * CHUNK)) * (NW * CHUNK)
    src_p = jnp.concatenate(
        [src, jnp.zeros((epad - e,), jnp.int32)]).reshape(NW, -1, CHUNK)
    dst_p = jnp.concatenate(
        [dst, jnp.full((epad - e,), N, jnp.int32)]).reshape(NW, -1, CHUNK)

    degp = _sc_deg(dst_p)
    dinv, g1 = _tc_prep(degp, x, W1)
    aggp1 = _sc_agg(g1, src_p, dst_p)
    g2 = _tc_mid(aggp1, g1, dinv, b1.reshape(1, D), W2)
    aggp2 = _sc_agg(g2, src_p, dst_p)
    return _tc_head(aggp2, g2, dinv, b2.reshape(1, D), Wfc, bfc.reshape(1, 1))
